# Initial kernel scaffold; baseline (speedup 1.0000x reference)
import jax
import jax.numpy as jnp
from jax.experimental import pallas as pl


def kernel(x, params):
    raise NotImplementedError("write your pallas kernel here")


# speedup vs baseline: 4.6146x; 4.6146x over previous
import functools

import jax
import jax.numpy as jnp
from jax.experimental import pallas as pl
from jax.experimental.pallas import tpu as pltpu
from jax.experimental.pallas import tpu_sc as plsc

B, C_IN, N, K = 8, 3, 2048, 20
OUT_CH = 256
_BN_S = 1.0 / (1.0 + 1e-5) ** 0.5
_NEG = float('-inf')


def _dot(a, w):
    return jax.lax.dot_general(
        a, w, (((a.ndim - 1,), (w.ndim - 1,)), ((), ())),
        preferred_element_type=jnp.float32,
        precision=jax.lax.Precision.HIGHEST)


def _pad2(w, o, i):
    return jnp.pad(w, ((0, o - w.shape[0]), (0, i - w.shape[1])))


def _pad1(v, o):
    return jnp.pad(v, (0, o - v.shape[0]))[None, :]


def _pointwise_body(x_ref, l01w, l01b, l02w, l02b, l11w, l11b, c2w, s2, t2,
                    xc1_ref, a_ref, x2_ref, outc_ref):
    x = x_ref[...]
    xc1 = _dot(x, l01w[...]) + l01b[...]
    a = _dot(x, l02w[...]) + l02b[...]
    x2 = _dot(xc1, l11w[...]) + l11b[...]
    outc = jnp.maximum(_dot(xc1, c2w[...]) * s2[...] + t2[...], 0.0)
    xc1_ref[...] = xc1
    a_ref[...] = a
    x2_ref[...] = x2
    outc_ref[...] = outc


_TW = 128


def _pointwise(xtp, p, cin_p, ca, dp, cout):
    bn = xtp.shape[0]
    blk = 512
    l01w = _pad2(p['l01_w'], ca, cin_p)
    l02w = _pad2(p['l02_w'], _TW, cin_p)
    l11w = _pad2(p['l11_w'], dp, ca)
    c2w = _pad2(p['c2_w'], cout, ca)
    l01b = _pad1(p['l01_b'], ca)
    l02b = _pad1(p['l02_b'], _TW)
    l11b = _pad1(p['l11_b'], dp)
    s2 = (p['bn2_g'] * _BN_S)[None, :]
    t2 = p['bn2_b'][None, :]
    wspec = lambda arr: pl.BlockSpec(arr.shape, lambda i: (0, 0))
    outs = (
        jax.ShapeDtypeStruct((bn, ca), jnp.float32),
        jax.ShapeDtypeStruct((bn, _TW), jnp.float32),
        jax.ShapeDtypeStruct((bn, dp), jnp.float32),
        jax.ShapeDtypeStruct((bn, cout), jnp.float32),
    )
    return pl.pallas_call(
        _pointwise_body,
        grid=(bn // blk,),
        in_specs=[pl.BlockSpec((blk, cin_p), lambda i: (i, 0))] +
                 [wspec(w) for w in (l01w, l01b, l02w, l02b, l11w, l11b, c2w, s2, t2)],
        out_specs=[pl.BlockSpec((blk, ca), lambda i: (i, 0)),
                   pl.BlockSpec((blk, _TW), lambda i: (i, 0)),
                   pl.BlockSpec((blk, dp), lambda i: (i, 0)),
                   pl.BlockSpec((blk, cout), lambda i: (i, 0))],
        out_shape=outs,
    )(xtp, l01w, l01b, l02w, l02b, l11w, l11b, c2w, s2, t2)


def _knn_body(xr_ref, xc_ref, idx_ref, *, rb):
    b = pl.program_id(0)
    xr = xr_ref[0]
    xc = xc_ref[0]
    inner = _dot(xr, xc)
    xxr = jnp.sum(xr * xr, axis=1, keepdims=True)
    xxc = _dot(jnp.ones((8, xc.shape[1]), jnp.float32), xc * xc)[0:1]
    pw = 2.0 * inner - xxr - xxc
    iota = jax.lax.broadcasted_iota(jnp.int32, (rb, N), 1)
    kiota = jax.lax.broadcasted_iota(jnp.int32, (rb, K), 1)
    acc = jnp.zeros((rb, K), jnp.int32)
    vals = pw
    base = b * N
    for t in range(K):
        m = jnp.max(vals, axis=1, keepdims=True)
        cand = jnp.where(vals == m, iota, N)
        j = jnp.min(cand, axis=1, keepdims=True)
        acc = jnp.where(kiota == t, j + base, acc)
        vals = jnp.where(cand == j, _NEG, vals)
    idx_ref[0] = acc


def _knn(xtp3):
    cp = xtp3.shape[2]
    rb = 256
    return pl.pallas_call(
        functools.partial(_knn_body, rb=rb),
        grid=(B, N // rb),
        in_specs=[pl.BlockSpec((1, rb, cp), lambda b, r: (b, r, 0)),
                  pl.BlockSpec((1, N, cp), lambda b, r: (b, 0, 0))],
        out_specs=pl.BlockSpec((1, rb, K), lambda b, r: (b, r, 0)),
        out_shape=jax.ShapeDtypeStruct((B, N, K), jnp.int32),
    )(xtp3, xtp3)


def _sc_gather(table, flat_idx):
    ni = flat_idx.shape[0]
    w = table.shape[1]
    gw = 128
    idx2 = flat_idx.reshape(1, ni)
    mesh = plsc.VectorSubcoreMesh(core_axis_name="core", subcore_axis_name="subcore")

    @pl.kernel(out_type=jax.ShapeDtypeStruct((ni, w), table.dtype), mesh=mesh)
    def gather_kernel(x_hbm, i_hbm, o_hbm):
        def body(i_vmem, o_vmem):
            pltpu.sync_copy(x_hbm.at[i_vmem.at[0]], o_vmem)

        pltpu.emit_pipeline(
            body,
            grid=(ni // gw,),
            in_specs=[pl.BlockSpec((1, gw), index_map=lambda i: (0, i))],
            out_specs=[pl.BlockSpec((gw, w), index_map=lambda i: (i, 0))],
            core_axis_name='subcore',
            dimension_semantics=(pltpu.PARALLEL,),
        )(i_hbm, o_hbm)

    return gather_kernel(table, idx2)


def _edge_body(ag_ref, xc1_ref, x2_ref, outc_ref, l12w, l12b, l2w, l2b,
               c1w, s1, t1, out_ref, *, pb, ca, dp, cout):
    ag = ag_ref[...].reshape(K * pb, _TW)[:, :ca]
    gb = _dot(ag, l12w[...]) + l12b[...]
    x2t = jnp.broadcast_to(x2_ref[0][None], (K, pb, dp)).reshape(K * pb, dp)
    s = x2t + gb
    t = jnp.where(s >= 0, s, 0.01 * s)
    c = _dot(t, l2w[...]) + l2b[...]
    xc1t = jnp.broadcast_to(xc1_ref[0][None], (K, pb, ca)).reshape(K * pb, ca)
    e = (ag - xc1t) * c
    y = _dot(e, c1w[...])
    z = jnp.maximum(y * s1[...] + t1[...], 0.0)
    zm = jnp.max(z.reshape(K, pb, cout), axis=0)
    out_ref[0] = outc_ref[0] + zm


def _edge(ag4, xc1, x2, outc, p, ca, dp, cout):
    pb = 128
    l12w = _pad2(p['l12_w'], dp, ca)
    l12b = _pad1(p['l12_b'], dp)
    l2w = _pad2(p['l2_w'], ca, dp)
    l2b = _pad1(p['l2_b'], ca)
    c1w = _pad2(p['c1_w'], cout, ca)
    s1 = (p['bn1_g'] * _BN_S)[None, :]
    t1 = p['bn1_b'][None, :]
    wspec = lambda arr: pl.BlockSpec(arr.shape, lambda b, n: (0, 0))
    return pl.pallas_call(
        functools.partial(_edge_body, pb=pb, ca=ca, dp=dp, cout=cout),
        grid=(B, N // pb),
        in_specs=[pl.BlockSpec((K, 1, pb, _TW), lambda b, n: (0, b, n, 0)),
                  pl.BlockSpec((1, pb, ca), lambda b, n: (b, n, 0)),
                  pl.BlockSpec((1, pb, dp), lambda b, n: (b, n, 0)),
                  pl.BlockSpec((1, pb, cout), lambda b, n: (b, n, 0))] +
                 [wspec(w) for w in (l12w, l12b, l2w, l2b, c1w, s1, t1)],
        out_specs=pl.BlockSpec((1, pb, cout), lambda b, n: (b, n, 0)),
        out_shape=jax.ShapeDtypeStruct((B, N, cout), jnp.float32),
    )(ag4, xc1, x2, outc, l12w, l12b, l2w, l2b, c1w, s1, t1)


def _edge_conv(xtp3, p, cin_p, c_real, cout):
    ca = 16 if c_real < 16 else c_real
    dp = 16 if c_real < 16 else 2 * c_real
    xc1, a, x2, outc = _pointwise(xtp3.reshape(B * N, cin_p), p, cin_p, ca, dp, cout)
    idx = _knn(xtp3)
    flat_idx = idx.transpose(2, 0, 1).reshape(-1)
    ag = _sc_gather(a, flat_idx)
    ag4 = ag.reshape(K, B, N, _TW)
    return _edge(ag4, xc1.reshape(B, N, ca), x2.reshape(B, N, dp),
                 outc.reshape(B, N, cout), p, ca, dp, cout)


def _convmax_body(x_ref, w, bias, g_ref):
    h = _dot(x_ref[0], w[...]) + bias[...]
    cur = jnp.max(h, axis=0, keepdims=True)[None]
    @pl.when(pl.program_id(1) == 0)
    def _():
        g_ref[...] = cur
    @pl.when(pl.program_id(1) != 0)
    def _():
        g_ref[...] = jnp.maximum(g_ref[...], cur)


def _convmax(x4, w, bias):
    nb = 512
    return pl.pallas_call(
        _convmax_body,
        grid=(B, N // nb),
        in_specs=[pl.BlockSpec((1, nb, 192), lambda b, n: (b, n, 0)),
                  pl.BlockSpec(w.shape, lambda b, n: (0, 0)),
                  pl.BlockSpec((1, 1024), lambda b, n: (0, 0))],
        out_specs=pl.BlockSpec((1, 1, 1024), lambda b, n: (b, 0, 0)),
        out_shape=jax.ShapeDtypeStruct((B, 1, 1024), jnp.float32),
    )(x4, w, bias)


def _tail_body(x_ref, g_ref, w1a, w1g, b1, s1, t1, w2, b2, s2, t2,
               w3, b3, s3, t3, w4, b4, out_ref):
    x = x_ref[0]
    h = _dot(x, w1a[...]) + _dot(g_ref[0], w1g[...]) + b1[...]
    h = h * s1[...] + t1[...]
    h = (_dot(h, w2[...]) + b2[...]) * s2[...] + t2[...]
    h = (_dot(h, w3[...]) + b3[...]) * s3[...] + t3[...]
    out_ref[0] = _dot(h, w4[...]) + b4[...]


def _tail(x4, g, params):
    nb = 256
    w1 = params['mlp1_w']
    args = (w1[:, :192], w1[:, 192:], params['mlp1_b'][None],
            (params['mlp1_g'] * _BN_S)[None], params['mlp1_be'][None],
            params['mlp2_w'], params['mlp2_b'][None],
            (params['mlp2_g'] * _BN_S)[None], params['mlp2_be'][None],
            params['mlp3_w'], params['mlp3_b'][None],
            (params['mlp3_g'] * _BN_S)[None], params['mlp3_be'][None],
            params['mlp4_w'], params['mlp4_b'][None])
    wspec = lambda arr: pl.BlockSpec(arr.shape, lambda b, n: (0, 0))
    return pl.pallas_call(
        _tail_body,
        grid=(B, N // nb),
        in_specs=[pl.BlockSpec((1, nb, 192), lambda b, n: (b, n, 0)),
                  pl.BlockSpec((1, 1, 1024), lambda b, n: (b, 0, 0))] +
                 [wspec(a) for a in args],
        out_specs=pl.BlockSpec((1, nb, OUT_CH), lambda b, n: (b, n, 0)),
        out_shape=jax.ShapeDtypeStruct((B, N, OUT_CH), jnp.float32),
    )(x4, g, *args)


def kernel(x, params):
    xt = jnp.transpose(x, (0, 2, 1))
    xtp = jnp.pad(xt, ((0, 0), (0, 0), (0, 8 - C_IN)))
    x1 = _edge_conv(xtp, params['ec1'], 8, C_IN, 64)
    x2 = _edge_conv(x1, params['ec2'], 64, 64, 64)
    x3 = _edge_conv(x2, params['ec3'], 64, 64, 64)
    x4 = jnp.concatenate([x1, x2, x3], axis=2)
    g = _convmax(x4, params['conv_w'], params['conv_b'][None])
    return _tail(x4, g, params)


# speedup vs baseline: 7.6857x; 1.6655x over previous
import functools

import jax
import jax.numpy as jnp
from jax.experimental import pallas as pl
from jax.experimental.pallas import tpu as pltpu
from jax.experimental.pallas import tpu_sc as plsc

B, C_IN, N, K = 8, 3, 2048, 20
OUT_CH = 256
_BN_S = 1.0 / (1.0 + 1e-5) ** 0.5
_NEG = float('-inf')


def _dot(a, w, prec=jax.lax.Precision.HIGHEST):
    return jax.lax.dot_general(
        a, w, (((a.ndim - 1,), (w.ndim - 1,)), ((), ())),
        preferred_element_type=jnp.float32, precision=prec)


def _dot3(a, w):
    ah = a.astype(jnp.bfloat16)
    al = (a - ah.astype(jnp.float32)).astype(jnp.bfloat16)
    wh = w.astype(jnp.bfloat16)
    wl = (w - wh.astype(jnp.float32)).astype(jnp.bfloat16)
    dn = (((a.ndim - 1,), (w.ndim - 1,)), ((), ()))
    kw = dict(preferred_element_type=jnp.float32,
              precision=jax.lax.Precision.DEFAULT)
    return (jax.lax.dot_general(ah, wh, dn, **kw) +
            (jax.lax.dot_general(al, wh, dn, **kw) +
             jax.lax.dot_general(ah, wl, dn, **kw)))


def _pad2(w, o, i):
    return jnp.pad(w, ((0, o - w.shape[0]), (0, i - w.shape[1])))


def _pad1(v, o):
    return jnp.pad(v, (0, o - v.shape[0]))[None, :]


def _pointwise_body(x_ref, l01w, l01b, l02w, l02b, l11w, l11b, c2w, s2, t2,
                    xc1_ref, a_ref, x2_ref, outc_ref):
    x = x_ref[...]
    xc1 = _dot3(x, l01w[...]) + l01b[...]
    a = _dot3(x, l02w[...]) + l02b[...]
    x2 = _dot3(xc1, l11w[...]) + l11b[...]
    outc = jnp.maximum(_dot3(xc1, c2w[...]) * s2[...] + t2[...], 0.0)
    xc1_ref[...] = xc1
    a_ref[...] = a
    x2_ref[...] = x2
    outc_ref[...] = outc


_TW = 128


def _pointwise(xtp, p, cin_p, ca, dp, cout):
    bn = xtp.shape[0]
    blk = 512
    l01w = _pad2(p['l01_w'], ca, cin_p)
    l02w = _pad2(p['l02_w'], _TW, cin_p)
    l11w = _pad2(p['l11_w'], dp, ca)
    c2w = _pad2(p['c2_w'], cout, ca)
    l01b = _pad1(p['l01_b'], ca)
    l02b = _pad1(p['l02_b'], _TW)
    l11b = _pad1(p['l11_b'], dp)
    s2 = (p['bn2_g'] * _BN_S)[None, :]
    t2 = p['bn2_b'][None, :]
    wspec = lambda arr: pl.BlockSpec(arr.shape, lambda i: (0, 0))
    outs = (
        jax.ShapeDtypeStruct((bn, ca), jnp.float32),
        jax.ShapeDtypeStruct((bn, _TW), jnp.float32),
        jax.ShapeDtypeStruct((bn, dp), jnp.float32),
        jax.ShapeDtypeStruct((bn, cout), jnp.float32),
    )
    return pl.pallas_call(
        _pointwise_body,
        grid=(bn // blk,),
        in_specs=[pl.BlockSpec((blk, cin_p), lambda i: (i, 0))] +
                 [wspec(w) for w in (l01w, l01b, l02w, l02b, l11w, l11b, c2w, s2, t2)],
        out_specs=[pl.BlockSpec((blk, ca), lambda i: (i, 0)),
                   pl.BlockSpec((blk, _TW), lambda i: (i, 0)),
                   pl.BlockSpec((blk, dp), lambda i: (i, 0)),
                   pl.BlockSpec((blk, cout), lambda i: (i, 0))],
        out_shape=outs,
    )(xtp, l01w, l01b, l02w, l02b, l11w, l11b, c2w, s2, t2)


def _knn_body(xr_ref, xc_ref, idx_ref, *, rb, boff):
    b = pl.program_id(0) + boff
    xr = xr_ref[0]
    xc = xc_ref[0]
    inner = _dot(xr, xc, jax.lax.Precision.HIGHEST)
    xxr = jnp.sum(xr * xr, axis=1, keepdims=True)
    xxc = _dot(jnp.ones((8, xc.shape[1]), jnp.float32), xc * xc,
               jax.lax.Precision.HIGHEST)[0:1]
    pw = 2.0 * inner - xxr - xxc
    iota = jax.lax.broadcasted_iota(jnp.int32, (rb, N), 1)
    kiota = jax.lax.broadcasted_iota(jnp.int32, (rb, K), 1)
    base = b * N
    rows = (pl.program_id(1) * rb
            + jax.lax.broadcasted_iota(jnp.int32, (rb, 1), 0))
    acc = jnp.broadcast_to(rows + base, (rb, K)).astype(jnp.int32)
    vals = jnp.where(iota == rows, _NEG, pw)
    for t in range(1, K):
        j = jnp.argmax(vals, axis=1)[:, None].astype(jnp.int32)
        acc = jnp.where(kiota == t, j + base, acc)
        vals = jnp.where(iota == j, _NEG, vals)
    idx_ref[0] = acc


def _knn(xtp3, boff=0):
    bb = xtp3.shape[0]
    cp = xtp3.shape[2]
    rb = 512
    return pl.pallas_call(
        functools.partial(_knn_body, rb=rb, boff=boff),
        grid=(bb, N // rb),
        in_specs=[pl.BlockSpec((1, rb, cp), lambda b, r: (b, r, 0)),
                  pl.BlockSpec((1, N, cp), lambda b, r: (b, 0, 0))],
        out_specs=pl.BlockSpec((1, rb, K), lambda b, r: (b, r, 0)),
        out_shape=jax.ShapeDtypeStruct((bb, N, K), jnp.int32),
    )(xtp3, xtp3)


def _sc_gather(table, flat_idx):
    ni = flat_idx.shape[0]
    w = table.shape[1]
    gw = 128
    idx2 = flat_idx.reshape(1, ni)
    mesh = plsc.VectorSubcoreMesh(core_axis_name="core", subcore_axis_name="subcore")

    @pl.kernel(out_type=jax.ShapeDtypeStruct((ni, w), table.dtype), mesh=mesh)
    def gather_kernel(x_hbm, i_hbm, o_hbm):
        def body(i_vmem, o_vmem):
            pltpu.sync_copy(x_hbm.at[i_vmem.at[0]], o_vmem)

        pltpu.emit_pipeline(
            body,
            grid=(ni // gw,),
            in_specs=[pl.BlockSpec((1, gw), index_map=lambda i: (0, i))],
            out_specs=[pl.BlockSpec((gw, w), index_map=lambda i: (i, 0))],
            core_axis_name=('core', 'subcore'),
            dimension_semantics=(pltpu.PARALLEL,),
        )(i_hbm, o_hbm)

    return gather_kernel(table, idx2)


def _edge_common(ag, xc1_ref, x2_ref, outc_ref, l12w, l12b, l2w, l2b,
                 c1w, s1, t1, out_ref, pb, ca, dp, cout):
    gb = _dot3(ag, l12w[...]) + l12b[...]
    x2t = jnp.broadcast_to(x2_ref[0][None], (K, pb, dp)).reshape(K * pb, dp)
    s = x2t + gb
    t = jnp.where(s >= 0, s, 0.01 * s)
    c = _dot3(t, l2w[...]) + l2b[...]
    xc1t = jnp.broadcast_to(xc1_ref[0][None], (K, pb, ca)).reshape(K * pb, ca)
    e = (ag - xc1t) * c
    y = _dot3(e, c1w[...])
    z = jnp.maximum(y * s1[...] + t1[...], 0.0)
    zm = jnp.max(z.reshape(K, pb, cout), axis=0)
    out_ref[0] = outc_ref[0] + zm


def _edge_body(ag_ref, xc1_ref, x2_ref, outc_ref, l12w, l12b, l2w, l2b,
               c1w, s1, t1, out_ref, *, pb, ca, dp, cout):
    ag = ag_ref[...].reshape(K * pb, _TW)[:, :ca]
    _edge_common(ag, xc1_ref, x2_ref, outc_ref, l12w, l12b, l2w, l2b,
                 c1w, s1, t1, out_ref, pb, ca, dp, cout)


def _edge(ag4, xc1, x2, outc, p, ca, dp, cout):
    bb = xc1.shape[0]
    pb = 512
    l12w = _pad2(p['l12_w'], dp, ca)
    l12b = _pad1(p['l12_b'], dp)
    l2w = _pad2(p['l2_w'], ca, dp)
    l2b = _pad1(p['l2_b'], ca)
    c1w = _pad2(p['c1_w'], cout, ca)
    s1 = (p['bn1_g'] * _BN_S)[None, :]
    t1 = p['bn1_b'][None, :]
    wspec = lambda arr: pl.BlockSpec(arr.shape, lambda b, n: (0, 0))
    return pl.pallas_call(
        functools.partial(_edge_body, pb=pb, ca=ca, dp=dp, cout=cout),
        grid=(bb, N // pb),
        in_specs=[pl.BlockSpec((K, 1, pb, _TW), lambda b, n: (0, b, n, 0)),
                  pl.BlockSpec((1, pb, ca), lambda b, n: (b, n, 0)),
                  pl.BlockSpec((1, pb, dp), lambda b, n: (b, n, 0)),
                  pl.BlockSpec((1, pb, cout), lambda b, n: (b, n, 0))] +
                 [wspec(w) for w in (l12w, l12b, l2w, l2b, c1w, s1, t1)],
        out_specs=pl.BlockSpec((1, pb, cout), lambda b, n: (b, n, 0)),
        out_shape=jax.ShapeDtypeStruct((bb, N, cout), jnp.float32),
    )(ag4, xc1, x2, outc, l12w, l12b, l2w, l2b, c1w, s1, t1)


def _edge_conv(xtp3, p, cin_p, c_real, cout):
    ca = 16 if c_real < 16 else c_real
    dp = 16 if c_real < 16 else 2 * c_real
    xc1, a, x2, outc = _pointwise(xtp3.reshape(B * N, cin_p), p, cin_p, ca, dp, cout)
    xc13 = xc1.reshape(B, N, ca)
    x23 = x2.reshape(B, N, dp)
    outc3 = outc.reshape(B, N, cout)
    h = B // 4
    outs = []
    for lo, hi in ((0, h), (h, 2 * h), (2 * h, 3 * h), (3 * h, B)):
        idx = _knn(xtp3[lo:hi], lo)
        ag = _sc_gather(a, idx.transpose(2, 0, 1).reshape(-1))
        ag4 = ag.reshape(K, hi - lo, N, _TW)
        outs.append(_edge(ag4, xc13[lo:hi], x23[lo:hi], outc3[lo:hi],
                          p, ca, dp, cout))
    return jnp.concatenate(outs, axis=0)


def _convmax_body(x_ref, w, bias, g_ref):
    h = _dot3(x_ref[0], w[...]) + bias[...]
    cur = jnp.max(h, axis=0, keepdims=True)[None]
    @pl.when(pl.program_id(1) == 0)
    def _():
        g_ref[...] = cur
    @pl.when(pl.program_id(1) != 0)
    def _():
        g_ref[...] = jnp.maximum(g_ref[...], cur)


def _convmax(x4, w, bias):
    nb = 512
    return pl.pallas_call(
        _convmax_body,
        grid=(B, N // nb),
        in_specs=[pl.BlockSpec((1, nb, 192), lambda b, n: (b, n, 0)),
                  pl.BlockSpec(w.shape, lambda b, n: (0, 0)),
                  pl.BlockSpec((1, 1024), lambda b, n: (0, 0))],
        out_specs=pl.BlockSpec((1, 1, 1024), lambda b, n: (b, 0, 0)),
        out_shape=jax.ShapeDtypeStruct((B, 1, 1024), jnp.float32),
    )(x4, w, bias)


def _tail_body(x_ref, g_ref, w1a, w1g, b1, s1, t1, w2, b2, s2, t2,
               w3, b3, s3, t3, w4, b4, out_ref):
    x = x_ref[0]
    h = _dot3(x, w1a[...]) + _dot3(g_ref[0], w1g[...]) + b1[...]
    h = h * s1[...] + t1[...]
    h = (_dot3(h, w2[...]) + b2[...]) * s2[...] + t2[...]
    h = (_dot3(h, w3[...]) + b3[...]) * s3[...] + t3[...]
    out_ref[0] = _dot3(h, w4[...]) + b4[...]


def _tail(x4, g, params):
    nb = 256
    w1 = params['mlp1_w']
    args = (w1[:, :192], w1[:, 192:], params['mlp1_b'][None],
            (params['mlp1_g'] * _BN_S)[None], params['mlp1_be'][None],
            params['mlp2_w'], params['mlp2_b'][None],
            (params['mlp2_g'] * _BN_S)[None], params['mlp2_be'][None],
            params['mlp3_w'], params['mlp3_b'][None],
            (params['mlp3_g'] * _BN_S)[None], params['mlp3_be'][None],
            params['mlp4_w'], params['mlp4_b'][None])
    wspec = lambda arr: pl.BlockSpec(arr.shape, lambda b, n: (0, 0))
    return pl.pallas_call(
        _tail_body,
        grid=(B, N // nb),
        in_specs=[pl.BlockSpec((1, nb, 192), lambda b, n: (b, n, 0)),
                  pl.BlockSpec((1, 1, 1024), lambda b, n: (b, 0, 0))] +
                 [wspec(a) for a in args],
        out_specs=pl.BlockSpec((1, nb, OUT_CH), lambda b, n: (b, n, 0)),
        out_shape=jax.ShapeDtypeStruct((B, N, OUT_CH), jnp.float32),
    )(x4, g, *args)


def kernel(x, params):
    xt = jnp.transpose(x, (0, 2, 1))
    xtp = jnp.pad(xt, ((0, 0), (0, 0), (0, 8 - C_IN)))
    x1 = _edge_conv(xtp, params['ec1'], 8, C_IN, 64)
    x2 = _edge_conv(x1, params['ec2'], 64, 64, 64)
    x3 = _edge_conv(x2, params['ec3'], 64, 64, 64)
    x4 = jnp.concatenate([x1, x2, x3], axis=2)
    g = _convmax(x4, params['conv_w'], params['conv_b'][None])
    return _tail(x4, g, params)
